# Initial kernel scaffold; baseline (speedup 1.0000x reference)
import jax
import jax.numpy as jnp
from jax.experimental import pallas as pl


def kernel(x, emb_tables, W_diag, b_diag, W_proc, b_proc):
    raise NotImplementedError("write your pallas kernel here")


# speedup vs baseline: 2.6252x; 2.6252x over previous
import functools

import jax
import jax.numpy as jnp
from jax import lax
from jax.experimental import pallas as pl
from jax.experimental.pallas import tpu as pltpu
from jax.experimental.pallas import tpu_sc as plsc

B = 16384
PN = 26
V = 100
D = 128
OUT = 1024
IN = PN * D

NC = 2
NS = 16
NW = NC * NS
BC = B // NW


def _sc_gather(xt_off: jax.Array, emb_flat: jax.Array) -> jax.Array:
    mesh = plsc.VectorSubcoreMesh(core_axis_name="c", subcore_axis_name="s")

    @functools.partial(
        pl.kernel,
        out_type=jax.ShapeDtypeStruct((B, IN), jnp.float32),
        mesh=mesh,
        scratch_types=[
            pltpu.VMEM((BC,), jnp.int32),
            pltpu.VMEM((BC, D), jnp.float32),
            pltpu.SemaphoreType.DMA,
        ],
    )
    def k(xt_hbm, emb_hbm, pe_hbm, idx_v, rows_v, sem):
        wid = lax.axis_index("s") * NC + lax.axis_index("c")
        base = wid * BC

        def body(i, carry):
            pltpu.sync_copy(xt_hbm.at[i, pl.ds(base, BC)], idx_v)
            pltpu.async_copy(emb_hbm.at[idx_v], rows_v, sem).wait()
            pltpu.sync_copy(rows_v, pe_hbm.at[pl.ds(base, BC), pl.ds(i * D, D)])
            return carry

        lax.fori_loop(0, PN, body, 0)

    return k(xt_off, emb_flat)


def _tc_heads(pe, wd_t, wp_t, bd, bp):
    BB = 256
    nb = B // BB

    def mm(pe_ref, wd_ref, wp_ref, bd_ref, bp_ref, dg_ref, pc_ref):
        a = pe_ref[...].astype(jnp.bfloat16)
        dg_ref[...] = (
            jnp.dot(a, wd_ref[...], preferred_element_type=jnp.float32)
            + bd_ref[...]
        )
        pc_ref[...] = (
            jnp.dot(a, wp_ref[...], preferred_element_type=jnp.float32)
            + bp_ref[...]
        )

    return pl.pallas_call(
        mm,
        grid=(nb,),
        in_specs=[
            pl.BlockSpec((BB, IN), lambda b: (b, 0)),
            pl.BlockSpec((IN, OUT), lambda b: (0, 0)),
            pl.BlockSpec((IN, OUT), lambda b: (0, 0)),
            pl.BlockSpec((1, OUT), lambda b: (0, 0)),
            pl.BlockSpec((1, OUT), lambda b: (0, 0)),
        ],
        out_specs=[
            pl.BlockSpec((BB, OUT), lambda b: (b, 0)),
            pl.BlockSpec((BB, OUT), lambda b: (b, 0)),
        ],
        out_shape=[
            jax.ShapeDtypeStruct((B, OUT), jnp.float32),
            jax.ShapeDtypeStruct((B, OUT), jnp.float32),
        ],
    )(pe, wd_t, wp_t, bd, bp)


def kernel(x, emb_tables, W_diag, b_diag, W_proc, b_proc):
    xt_off = (
        x.astype(jnp.int32) + V * jnp.arange(PN, dtype=jnp.int32)[None, :]
    ).T
    emb_flat = emb_tables.reshape(PN * V, D)

    pe = _sc_gather(xt_off, emb_flat)

    wd_t = W_diag.T.astype(jnp.bfloat16)
    wp_t = W_proc.T.astype(jnp.bfloat16)
    diag, proc = _tc_heads(
        pe, wd_t, wp_t,
        b_diag.reshape(1, OUT), b_proc.reshape(1, OUT),
    )
    return (diag, proc, diag)


# speedup vs baseline: 2.7550x; 1.0494x over previous
import functools

import jax
import jax.numpy as jnp
from jax import lax
from jax.experimental import pallas as pl
from jax.experimental.pallas import tpu as pltpu
from jax.experimental.pallas import tpu_sc as plsc

B = 16384
PN = 26
V = 100
D = 128
OUT = 1024
IN = PN * D

NC = 2
NS = 16
NW = NC * NS
BC = B // NW


SUB = 128
NSUB = BC // SUB


def _sc_gather(xt_off: jax.Array, emb_flat: jax.Array) -> jax.Array:
    mesh = plsc.VectorSubcoreMesh(core_axis_name="c", subcore_axis_name="s")

    @functools.partial(
        pl.kernel,
        out_type=jax.ShapeDtypeStruct((B, IN), jnp.float32),
        mesh=mesh,
        scratch_types=[
            pltpu.VMEM((PN, BC), jnp.int32),
            *[pltpu.VMEM((SUB, D), jnp.float32) for _ in range(4)],
            *[pltpu.SemaphoreType.DMA for _ in range(8)],
        ],
    )
    def k(xt_hbm, emb_hbm, pe_hbm, idx2,
          b0, b1, b2, b3, g0, g1, g2, g3, w0, w1, w2, w3):
        bufs = (b0, b1, b2, b3)
        gs = (g0, g1, g2, g3)
        ws = (w0, w1, w2, w3)
        wid = lax.axis_index("s") * NC + lax.axis_index("c")
        base = wid * BC

        pltpu.sync_copy(xt_hbm.at[:, pl.ds(base, BC)], idx2)

        def gather_start(i, s, p):
            pltpu.async_copy(
                emb_hbm.at[idx2.at[i, pl.ds(s * SUB, SUB)]], bufs[p], gs[p])

        def gather_wait(p):
            pltpu.make_async_copy(
                emb_hbm.at[pl.ds(0, SUB)], bufs[p], gs[p]).wait()

        def write_start(i, s, p):
            pltpu.async_copy(
                bufs[p],
                pe_hbm.at[pl.ds(base + s * SUB, SUB), pl.ds(i * D, D)],
                ws[p])

        def write_wait(p):
            pltpu.make_async_copy(
                bufs[p],
                pe_hbm.at[pl.ds(0, SUB), pl.ds(0, D)],
                ws[p]).wait()

        gather_start(0, 0, 0)
        gather_start(0, 1, 1)
        for p in range(4):
            gather_wait(p)
            write_start(0, p, p)
            if p >= 2:
                write_wait((p + 2) % 4)
            gather_start((p + 2) // 4, (p + 2) % 4, (p + 2) % 4)

        def body(j, carry):
            for p in range(4):
                gather_wait(p)
                write_start(j, p, p)
                write_wait((p + 2) % 4)
                gather_start(j + (1 if p >= 2 else 0), (p + 2) % 4,
                             (p + 2) % 4)
            return carry

        lax.fori_loop(1, PN - 1, body, 0)

        for p in range(4):
            gather_wait(p)
            write_start(PN - 1, p, p)
            if p < 2:
                write_wait((p + 2) % 4)
                gather_start(PN - 1, p + 2, p + 2)
        for p in range(4):
            write_wait(p)

    return k(xt_off, emb_flat)


def _tc_heads(pe, wd_t, wp_t, bd, bp):
    BB = 512
    nb = B // BB

    def mm(pe_ref, wd_ref, wp_ref, bd_ref, bp_ref, dg_ref, pc_ref):
        a = pe_ref[...].astype(jnp.bfloat16)
        dg_ref[...] = (
            jnp.dot(a, wd_ref[...], preferred_element_type=jnp.float32)
            + bd_ref[...]
        )
        pc_ref[...] = (
            jnp.dot(a, wp_ref[...], preferred_element_type=jnp.float32)
            + bp_ref[...]
        )

    return pl.pallas_call(
        mm,
        grid=(nb,),
        in_specs=[
            pl.BlockSpec((BB, IN), lambda b: (b, 0)),
            pl.BlockSpec((IN, OUT), lambda b: (0, 0)),
            pl.BlockSpec((IN, OUT), lambda b: (0, 0)),
            pl.BlockSpec((1, OUT), lambda b: (0, 0)),
            pl.BlockSpec((1, OUT), lambda b: (0, 0)),
        ],
        out_specs=[
            pl.BlockSpec((BB, OUT), lambda b: (b, 0)),
            pl.BlockSpec((BB, OUT), lambda b: (b, 0)),
        ],
        out_shape=[
            jax.ShapeDtypeStruct((B, OUT), jnp.float32),
            jax.ShapeDtypeStruct((B, OUT), jnp.float32),
        ],
    )(pe, wd_t, wp_t, bd, bp)


def kernel(x, emb_tables, W_diag, b_diag, W_proc, b_proc):
    xt_off = (
        x.astype(jnp.int32) + V * jnp.arange(PN, dtype=jnp.int32)[None, :]
    ).T
    emb_flat = emb_tables.reshape(PN * V, D)

    pe = _sc_gather(xt_off, emb_flat)

    wd_t = W_diag.T.astype(jnp.bfloat16)
    wp_t = W_proc.T.astype(jnp.bfloat16)
    diag, proc = _tc_heads(
        pe, wd_t, wp_t,
        b_diag.reshape(1, OUT), b_proc.reshape(1, OUT),
    )
    return (diag, proc, diag)


# speedup vs baseline: 3.0236x; 1.0975x over previous
import functools

import jax
import jax.numpy as jnp
from jax import lax
from jax.experimental import pallas as pl
from jax.experimental.pallas import tpu as pltpu
from jax.experimental.pallas import tpu_sc as plsc

B = 16384
PN = 26
V = 100
D = 128
OUT = 1024
IN = PN * D

NC = 2
NS = 16
NW = NC * NS

NCH = 4
CH = B // NCH
BCW = CH // NW


def _sc_gather_chunk(xt_c: jax.Array, emb_flat: jax.Array) -> jax.Array:
    mesh = plsc.VectorSubcoreMesh(core_axis_name="c", subcore_axis_name="s")

    @functools.partial(
        pl.kernel,
        out_type=jax.ShapeDtypeStruct((CH, IN), jnp.float32),
        mesh=mesh,
        scratch_types=[
            pltpu.VMEM((PN, BCW), jnp.int32),
            *[pltpu.VMEM((BCW, D), jnp.float32) for _ in range(4)],
            *[pltpu.SemaphoreType.DMA for _ in range(8)],
        ],
    )
    def k(xt_hbm, emb_hbm, pe_hbm, idx2,
          b0, b1, b2, b3, g0, g1, g2, g3, w0, w1, w2, w3):
        bufs = (b0, b1, b2, b3)
        gs = (g0, g1, g2, g3)
        ws = (w0, w1, w2, w3)
        wid = lax.axis_index("s") * NC + lax.axis_index("c")
        base = wid * BCW

        pltpu.sync_copy(xt_hbm.at[:, pl.ds(base, BCW)], idx2)

        def gather_start(i, p):
            pltpu.async_copy(emb_hbm.at[idx2.at[i]], bufs[p], gs[p])

        def gather_wait(p):
            pltpu.make_async_copy(
                emb_hbm.at[pl.ds(0, BCW)], bufs[p], gs[p]).wait()

        def write_start(i, p):
            pltpu.async_copy(
                bufs[p],
                pe_hbm.at[pl.ds(base, BCW), pl.ds(i * D, D)],
                ws[p])

        def write_wait(p):
            pltpu.make_async_copy(
                bufs[p],
                pe_hbm.at[pl.ds(0, BCW), pl.ds(0, D)],
                ws[p]).wait()

        gather_start(0, 0)
        gather_start(1, 1)
        for t in (0, 1):
            gather_wait(t)
            write_start(t, t)
            gather_start(t + 2, (t + 2) % 4)

        def body(j, carry):
            for p in range(4):
                t = 4 * j + 2 + p
                q = (p + 2) % 4
                gather_wait(q)
                write_start(t, q)
                write_wait(p)
                gather_start(t + 2, p)
            return carry

        lax.fori_loop(0, 5, body, 0)

        for t in (22, 23):
            q = t % 4
            gather_wait(q)
            write_start(t, q)
            write_wait((q + 2) % 4)
            gather_start(t + 2, (q + 2) % 4)
        for t in (24, 25):
            q = t % 4
            gather_wait(q)
            write_start(t, q)
        for p in range(4):
            write_wait(p)

    return k(xt_c, emb_flat)


def _tc_heads(pe, wd_t, wp_t, bd, bp):
    BB = 512
    nb = CH // BB

    def mm(pe_ref, wd_ref, wp_ref, bd_ref, bp_ref, dg_ref, pc_ref):
        a = pe_ref[...].astype(jnp.bfloat16)
        dg_ref[...] = (
            jnp.dot(a, wd_ref[...], preferred_element_type=jnp.float32)
            + bd_ref[...]
        )
        pc_ref[...] = (
            jnp.dot(a, wp_ref[...], preferred_element_type=jnp.float32)
            + bp_ref[...]
        )

    return pl.pallas_call(
        mm,
        grid=(nb,),
        in_specs=[
            pl.BlockSpec((BB, IN), lambda b: (b, 0)),
            pl.BlockSpec((IN, OUT), lambda b: (0, 0)),
            pl.BlockSpec((IN, OUT), lambda b: (0, 0)),
            pl.BlockSpec((1, OUT), lambda b: (0, 0)),
            pl.BlockSpec((1, OUT), lambda b: (0, 0)),
        ],
        out_specs=[
            pl.BlockSpec((BB, OUT), lambda b: (b, 0)),
            pl.BlockSpec((BB, OUT), lambda b: (b, 0)),
        ],
        out_shape=[
            jax.ShapeDtypeStruct((CH, OUT), jnp.float32),
            jax.ShapeDtypeStruct((CH, OUT), jnp.float32),
        ],
    )(pe, wd_t, wp_t, bd, bp)


def kernel(x, emb_tables, W_diag, b_diag, W_proc, b_proc):
    xt_off = (
        x.astype(jnp.int32) + V * jnp.arange(PN, dtype=jnp.int32)[None, :]
    ).T
    emb_flat = emb_tables.reshape(PN * V, D)

    wd_t = W_diag.T.astype(jnp.bfloat16)
    wp_t = W_proc.T.astype(jnp.bfloat16)
    bd = b_diag.reshape(1, OUT)
    bp = b_proc.reshape(1, OUT)

    outs = []
    for c in range(NCH):
        pe_c = _sc_gather_chunk(
            xt_off[:, c * CH:(c + 1) * CH], emb_flat)
        outs.append(_tc_heads(pe_c, wd_t, wp_t, bd, bp))

    diag = jnp.concatenate([o[0] for o in outs], axis=0)
    proc = jnp.concatenate([o[1] for o in outs], axis=0)
    return (diag, proc, diag)


# speedup vs baseline: 3.0953x; 1.0237x over previous
import functools

import jax
import jax.numpy as jnp
from jax import lax
from jax.experimental import pallas as pl
from jax.experimental.pallas import tpu as pltpu
from jax.experimental.pallas import tpu_sc as plsc

B = 16384
PN = 26
V = 100
D = 128
OUT = 1024
IN = PN * D

NC = 2
NS = 16
NW = NC * NS

NCH = 4
CH = B // NCH
BCW = CH // NW


def _sc_gather_chunk(xt_c: jax.Array, emb_flat: jax.Array) -> jax.Array:
    mesh = plsc.VectorSubcoreMesh(core_axis_name="c", subcore_axis_name="s")

    @functools.partial(
        pl.kernel,
        out_type=jax.ShapeDtypeStruct((CH, IN), jnp.float32),
        mesh=mesh,
        scratch_types=[
            pltpu.VMEM((PN, BCW), jnp.int32),
            *[pltpu.VMEM((BCW, D), jnp.float32) for _ in range(4)],
            *[pltpu.SemaphoreType.DMA for _ in range(8)],
        ],
    )
    def k(xt_hbm, emb_hbm, pe_hbm, idx2,
          b0, b1, b2, b3, g0, g1, g2, g3, w0, w1, w2, w3):
        bufs = (b0, b1, b2, b3)
        gs = (g0, g1, g2, g3)
        ws = (w0, w1, w2, w3)
        wid = lax.axis_index("s") * NC + lax.axis_index("c")
        base = wid * BCW

        pltpu.sync_copy(xt_hbm.at[:, pl.ds(base, BCW)], idx2)

        def gather_start(i, p):
            pltpu.async_copy(emb_hbm.at[idx2.at[i]], bufs[p], gs[p])

        def gather_wait(p):
            pltpu.make_async_copy(
                emb_hbm.at[pl.ds(0, BCW)], bufs[p], gs[p]).wait()

        def write_start(i, p):
            pltpu.async_copy(
                bufs[p],
                pe_hbm.at[pl.ds(base, BCW), pl.ds(i * D, D)],
                ws[p])

        def write_wait(p):
            pltpu.make_async_copy(
                bufs[p],
                pe_hbm.at[pl.ds(0, BCW), pl.ds(0, D)],
                ws[p]).wait()

        gather_start(0, 0)
        gather_start(1, 1)
        for t in (0, 1):
            gather_wait(t)
            write_start(t, t)
            gather_start(t + 2, (t + 2) % 4)

        def body(j, carry):
            for p in range(4):
                t = 4 * j + 2 + p
                q = (p + 2) % 4
                gather_wait(q)
                write_start(t, q)
                write_wait(p)
                gather_start(t + 2, p)
            return carry

        lax.fori_loop(0, 5, body, 0)

        for t in (22, 23):
            q = t % 4
            gather_wait(q)
            write_start(t, q)
            write_wait((q + 2) % 4)
            gather_start(t + 2, (q + 2) % 4)
        for t in (24, 25):
            q = t % 4
            gather_wait(q)
            write_start(t, q)
        for p in range(4):
            write_wait(p)

    return k(xt_c, emb_flat)


def _tc_heads_chunk(c, prev, pe_c, wd_t, wp_t, bd, bp):
    BB = 512
    nb = CH // BB

    def mm(dfull_ref, pfull_ref, pe_ref, wd_ref, wp_ref, bd_ref, bp_ref,
           dg_ref, pc_ref):
        del dfull_ref, pfull_ref
        a = pe_ref[...].astype(jnp.bfloat16)
        dg_ref[...] = (
            jnp.dot(a, wd_ref[...], preferred_element_type=jnp.float32)
            + bd_ref[...]
        )
        pc_ref[...] = (
            jnp.dot(a, wp_ref[...], preferred_element_type=jnp.float32)
            + bp_ref[...]
        )

    return pl.pallas_call(
        mm,
        grid=(nb,),
        in_specs=[
            pl.BlockSpec(memory_space=pl.ANY),
            pl.BlockSpec(memory_space=pl.ANY),
            pl.BlockSpec((BB, IN), lambda b: (b, 0)),
            pl.BlockSpec((IN, OUT), lambda b: (0, 0)),
            pl.BlockSpec((IN, OUT), lambda b: (0, 0)),
            pl.BlockSpec((1, OUT), lambda b: (0, 0)),
            pl.BlockSpec((1, OUT), lambda b: (0, 0)),
        ],
        out_specs=[
            pl.BlockSpec((BB, OUT), lambda b, _c=c, _nb=nb: (b + _c * _nb, 0)),
            pl.BlockSpec((BB, OUT), lambda b, _c=c, _nb=nb: (b + _c * _nb, 0)),
        ],
        out_shape=[
            jax.ShapeDtypeStruct((B, OUT), jnp.float32),
            jax.ShapeDtypeStruct((B, OUT), jnp.float32),
        ],
        input_output_aliases={0: 0, 1: 1},
    )(prev[0], prev[1], pe_c, wd_t, wp_t, bd, bp)


def kernel(x, emb_tables, W_diag, b_diag, W_proc, b_proc):
    xt_off = (
        x.astype(jnp.int32) + V * jnp.arange(PN, dtype=jnp.int32)[None, :]
    ).T
    emb_flat = emb_tables.reshape(PN * V, D)

    wd_t = W_diag.T.astype(jnp.bfloat16)
    wp_t = W_proc.T.astype(jnp.bfloat16)
    bd = b_diag.reshape(1, OUT)
    bp = b_proc.reshape(1, OUT)

    prev = (
        jnp.zeros((B, OUT), jnp.float32),
        jnp.zeros((B, OUT), jnp.float32),
    )
    for c in range(NCH):
        pe_c = _sc_gather_chunk(
            xt_off[:, c * CH:(c + 1) * CH], emb_flat)
        prev = _tc_heads_chunk(c, prev, pe_c, wd_t, wp_t, bd, bp)

    diag, proc = prev
    return (diag, proc, diag)


# speedup vs baseline: 3.5709x; 1.1536x over previous
import functools

import jax
import jax.numpy as jnp
from jax import lax
from jax.experimental import pallas as pl
from jax.experimental.pallas import tpu as pltpu
from jax.experimental.pallas import tpu_sc as plsc

B = 16384
PN = 26
V = 100
D = 128
OUT = 1024
IN = PN * D

NC = 2
NS = 16
NW = NC * NS

NCH = 4
CH = B // NCH
BCW = CH // NW


def _sc_gather_chunk(xt_c: jax.Array, emb_flat: jax.Array) -> jax.Array:
    mesh = plsc.VectorSubcoreMesh(core_axis_name="c", subcore_axis_name="s")

    @functools.partial(
        pl.kernel,
        out_type=jax.ShapeDtypeStruct((CH, IN), jnp.float32),
        mesh=mesh,
        scratch_types=[
            pltpu.VMEM((PN, BCW), jnp.int32),
            *[pltpu.VMEM((BCW, D), jnp.float32) for _ in range(4)],
            *[pltpu.SemaphoreType.DMA for _ in range(8)],
        ],
    )
    def k(xt_hbm, emb_hbm, pe_hbm, idx2,
          b0, b1, b2, b3, g0, g1, g2, g3, w0, w1, w2, w3):
        bufs = (b0, b1, b2, b3)
        gs = (g0, g1, g2, g3)
        ws = (w0, w1, w2, w3)
        wid = lax.axis_index("s") * NC + lax.axis_index("c")
        base = wid * BCW

        pltpu.sync_copy(xt_hbm.at[:, pl.ds(base, BCW)], idx2)

        def gather_start(i, p):
            pltpu.async_copy(emb_hbm.at[idx2.at[i]], bufs[p], gs[p])

        def gather_wait(p):
            pltpu.make_async_copy(
                emb_hbm.at[pl.ds(0, BCW)], bufs[p], gs[p]).wait()

        def write_start(i, p):
            pltpu.async_copy(
                bufs[p],
                pe_hbm.at[pl.ds(base, BCW), pl.ds(i * D, D)],
                ws[p])

        def write_wait(p):
            pltpu.make_async_copy(
                bufs[p],
                pe_hbm.at[pl.ds(0, BCW), pl.ds(0, D)],
                ws[p]).wait()

        gather_start(0, 0)
        gather_start(1, 1)
        for t in (0, 1):
            gather_wait(t)
            write_start(t, t)
            gather_start(t + 2, (t + 2) % 4)

        def body(j, carry):
            for p in range(4):
                t = 4 * j + 2 + p
                q = (p + 2) % 4
                gather_wait(q)
                write_start(t, q)
                write_wait(p)
                gather_start(t + 2, p)
            return carry

        lax.fori_loop(0, 5, body, 0)

        for t in (22, 23):
            q = t % 4
            gather_wait(q)
            write_start(t, q)
            write_wait((q + 2) % 4)
            gather_start(t + 2, (q + 2) % 4)
        for t in (24, 25):
            q = t % 4
            gather_wait(q)
            write_start(t, q)
        for p in range(4):
            write_wait(p)

    return k(xt_c, emb_flat)


def _tc_heads_chunk(c, prev, pe_c, wd_t, wp_t, bd, bp):
    BB = 512
    nb = CH // BB

    def mm(*refs):
        pe_ref, wd_ref, wp_ref, bd_ref, bp_ref, dg_ref, pc_ref = refs[-7:]
        a = pe_ref[...].astype(jnp.bfloat16)
        dg_ref[...] = (
            jnp.dot(a, wd_ref[...], preferred_element_type=jnp.float32)
            + bd_ref[...]
        )
        pc_ref[...] = (
            jnp.dot(a, wp_ref[...], preferred_element_type=jnp.float32)
            + bp_ref[...]
        )

    alias_specs = [] if c == 0 else [pl.BlockSpec(memory_space=pl.ANY)] * 2
    alias_args = () if c == 0 else (prev[0], prev[1])
    return pl.pallas_call(
        mm,
        grid=(nb,),
        in_specs=alias_specs + [
            pl.BlockSpec((BB, IN), lambda b: (b, 0)),
            pl.BlockSpec((IN, OUT), lambda b: (0, 0)),
            pl.BlockSpec((IN, OUT), lambda b: (0, 0)),
            pl.BlockSpec((1, OUT), lambda b: (0, 0)),
            pl.BlockSpec((1, OUT), lambda b: (0, 0)),
        ],
        out_specs=[
            pl.BlockSpec((BB, OUT), lambda b, _c=c, _nb=nb: (b + _c * _nb, 0)),
            pl.BlockSpec((BB, OUT), lambda b, _c=c, _nb=nb: (b + _c * _nb, 0)),
        ],
        out_shape=[
            jax.ShapeDtypeStruct((B, OUT), jnp.float32),
            jax.ShapeDtypeStruct((B, OUT), jnp.float32),
        ],
        input_output_aliases={} if c == 0 else {0: 0, 1: 1},
    )(*alias_args, pe_c, wd_t, wp_t, bd, bp)


def kernel(x, emb_tables, W_diag, b_diag, W_proc, b_proc):
    xt_off = (
        x.astype(jnp.int32) + V * jnp.arange(PN, dtype=jnp.int32)[None, :]
    ).T
    emb_flat = emb_tables.reshape(PN * V, D)

    wd_t = W_diag.T.astype(jnp.bfloat16)
    wp_t = W_proc.T.astype(jnp.bfloat16)
    bd = b_diag.reshape(1, OUT)
    bp = b_proc.reshape(1, OUT)

    prev = None
    for c in range(NCH):
        pe_c = _sc_gather_chunk(
            xt_off[:, c * CH:(c + 1) * CH], emb_flat)
        prev = _tc_heads_chunk(c, prev, pe_c, wd_t, wp_t, bd, bp)

    diag, proc = prev
    return (diag, proc, diag)


# speedup vs baseline: 3.7305x; 1.0447x over previous
import functools

import jax
import jax.numpy as jnp
from jax import lax
from jax.experimental import pallas as pl
from jax.experimental.pallas import tpu as pltpu
from jax.experimental.pallas import tpu_sc as plsc

B = 16384
PN = 26
V = 100
D = 128
OUT = 1024
IN = PN * D

NC = 2
NS = 16
NW = NC * NS

NCH = 4
CH = B // NCH
BCW = CH // NW


def _sc_gather_chunk(xt_c: jax.Array, emb_flat: jax.Array) -> jax.Array:
    mesh = plsc.VectorSubcoreMesh(core_axis_name="c", subcore_axis_name="s")

    @functools.partial(
        pl.kernel,
        out_type=jax.ShapeDtypeStruct((CH, IN), jnp.float32),
        mesh=mesh,
        scratch_types=[
            pltpu.VMEM((PN, BCW), jnp.int32),
            *[pltpu.VMEM((BCW, D), jnp.float32) for _ in range(4)],
            *[pltpu.SemaphoreType.DMA for _ in range(8)],
        ],
    )
    def k(xt_hbm, emb_hbm, pe_hbm, idx2,
          b0, b1, b2, b3, g0, g1, g2, g3, w0, w1, w2, w3):
        bufs = (b0, b1, b2, b3)
        gs = (g0, g1, g2, g3)
        ws = (w0, w1, w2, w3)
        wid = lax.axis_index("s") * NC + lax.axis_index("c")
        base = wid * BCW

        pltpu.sync_copy(xt_hbm.at[:, pl.ds(base, BCW)], idx2)

        def gather_start(i, p):
            pltpu.async_copy(emb_hbm.at[idx2.at[i]], bufs[p], gs[p])

        def gather_wait(p):
            pltpu.make_async_copy(
                emb_hbm.at[pl.ds(0, BCW)], bufs[p], gs[p]).wait()

        def write_start(i, p):
            pltpu.async_copy(
                bufs[p],
                pe_hbm.at[pl.ds(base, BCW), pl.ds(i * D, D)],
                ws[p])

        def write_wait(p):
            pltpu.make_async_copy(
                bufs[p],
                pe_hbm.at[pl.ds(0, BCW), pl.ds(0, D)],
                ws[p]).wait()

        gather_start(0, 0)
        gather_start(1, 1)
        for t in (0, 1):
            gather_wait(t)
            write_start(t, t)
            gather_start(t + 2, (t + 2) % 4)

        def body(j, carry):
            for p in range(4):
                t = 4 * j + 2 + p
                q = (p + 2) % 4
                gather_wait(q)
                write_start(t, q)
                write_wait(p)
                gather_start(t + 2, p)
            return carry

        lax.fori_loop(0, 5, body, 0)

        for t in (22, 23):
            q = t % 4
            gather_wait(q)
            write_start(t, q)
            write_wait((q + 2) % 4)
            gather_start(t + 2, (q + 2) % 4)
        for t in (24, 25):
            q = t % 4
            gather_wait(q)
            write_start(t, q)
        for p in range(4):
            write_wait(p)

    return k(xt_c, emb_flat)


def _tc_heads_chunk(c, prev, pe_c, wd_t, wp_t, bd, bp):
    BB = 512
    nb = CH // BB

    def mm(*refs):
        pe_ref, wd_ref, wp_ref, bd_ref, bp_ref = refs[-8:-3]
        dg_ref, pc_ref, dg2_ref = refs[-3:]
        a = pe_ref[...].astype(jnp.bfloat16)
        dg = (
            jnp.dot(a, wd_ref[...], preferred_element_type=jnp.float32)
            + bd_ref[...]
        )
        dg_ref[...] = dg
        dg2_ref[...] = dg
        pc_ref[...] = (
            jnp.dot(a, wp_ref[...], preferred_element_type=jnp.float32)
            + bp_ref[...]
        )

    alias_specs = [] if c == 0 else [pl.BlockSpec(memory_space=pl.ANY)] * 3
    alias_args = () if c == 0 else tuple(prev)
    out_spec = pl.BlockSpec(
        (BB, OUT), lambda b, _c=c, _nb=nb: (b + _c * _nb, 0))
    return pl.pallas_call(
        mm,
        grid=(nb,),
        in_specs=alias_specs + [
            pl.BlockSpec((BB, IN), lambda b: (b, 0)),
            pl.BlockSpec((IN, OUT), lambda b: (0, 0)),
            pl.BlockSpec((IN, OUT), lambda b: (0, 0)),
            pl.BlockSpec((1, OUT), lambda b: (0, 0)),
            pl.BlockSpec((1, OUT), lambda b: (0, 0)),
        ],
        out_specs=[out_spec, out_spec, out_spec],
        out_shape=[
            jax.ShapeDtypeStruct((B, OUT), jnp.float32),
            jax.ShapeDtypeStruct((B, OUT), jnp.float32),
            jax.ShapeDtypeStruct((B, OUT), jnp.float32),
        ],
        input_output_aliases={} if c == 0 else {0: 0, 1: 1, 2: 2},
    )(*alias_args, pe_c, wd_t, wp_t, bd, bp)


def kernel(x, emb_tables, W_diag, b_diag, W_proc, b_proc):
    xt_off = (
        x.astype(jnp.int32) + V * jnp.arange(PN, dtype=jnp.int32)[None, :]
    ).T
    emb_flat = emb_tables.reshape(PN * V, D)

    wd_t = W_diag.T.astype(jnp.bfloat16)
    wp_t = W_proc.T.astype(jnp.bfloat16)
    bd = b_diag.reshape(1, OUT)
    bp = b_proc.reshape(1, OUT)

    prev = None
    for c in range(NCH):
        pe_c = _sc_gather_chunk(
            xt_off[:, c * CH:(c + 1) * CH], emb_flat)
        prev = _tc_heads_chunk(c, prev, pe_c, wd_t, wp_t, bd, bp)

    diag, proc, diag2 = prev
    return (diag, proc, diag2)
